# Initial kernel scaffold; baseline (speedup 1.0000x reference)
import functools

import jax
import jax.numpy as jnp
from jax import lax
from jax.experimental import pallas as pl
from jax.experimental.pallas import tpu as pltpu
from jax.experimental.pallas import tpu_sc as plsc

_K = 16
_PW = 16
_TW = 80
_T = 1000
_CH = 128
_NW = 32


@functools.lru_cache(maxsize=None)
def _sc_gather_fn(V, D, Bn):
    n_chunks = Bn // _CH
    per_w = (n_chunks + _NW - 1) // _NW
    mesh = plsc.VectorSubcoreMesh(core_axis_name="c", subcore_axis_name="s")

    @functools.partial(
        pl.kernel,
        mesh=mesh,
        out_type=jax.ShapeDtypeStruct((Bn, D), jnp.float32),
        scratch_types=[
            pltpu.VMEM((_CH,), jnp.int32),
            pltpu.VMEM((_CH, D), jnp.float32),
            pltpu.SemaphoreType.DMA,
        ],
    )
    def gk(table_hbm, idx_hbm, out_hbm, idx_v, rows_v, sem):
        wid = lax.axis_index("s") * 2 + lax.axis_index("c")

        def body(t, carry):
            cidx = wid + t * _NW

            @pl.when(cidx < n_chunks)
            def _():
                base = cidx * _CH
                pltpu.sync_copy(idx_hbm.at[pl.ds(base, _CH)], idx_v)
                pltpu.async_copy(table_hbm.at[idx_v], rows_v, sem).wait()
                pltpu.sync_copy(rows_v, out_hbm.at[pl.ds(base, _CH)])

            return carry

        lax.fori_loop(0, per_w, body, 0)

    return gk


def _sc_gather(table, idx_flat):
    V, D = table.shape
    return _sc_gather_fn(V, D, idx_flat.shape[0])(table, idx_flat)


def _leaky(x):
    return jnp.where(x > 0, x, 0.2 * x)


def _elu(x):
    return jnp.where(x > 0, x, jnp.exp(jnp.minimum(x, 0.0)) - 1.0)


def _attn_pool(h, T, a1_ref, a2_ref, attb_ref):
    h3 = h.reshape(T, _K, 64)
    a1 = a1_ref[...].reshape(1, 1, 64)
    a2 = a2_ref[...]
    l1 = jnp.sum(h3 * a1, axis=-1)
    hc = h3[:, 0, :]
    l2 = jnp.sum(hc * a2, axis=-1, keepdims=True)
    logit = _leaky(l1 + l2 + attb_ref[0, 0])
    m = jnp.max(logit, axis=-1, keepdims=True)
    e = jnp.exp(logit - m)
    a = e / jnp.sum(e, axis=-1, keepdims=True)
    return _elu(jnp.sum(h3 * a[:, :, None], axis=1))


def _layer0_body(g_ref, c_ref, wps_ref, wc_ref, wd_ref, b0_ref,
                 w1_ref, b1_ref, w2_ref, b2_ref, a1_ref, a2_ref, attb_ref,
                 out_ref):
    T = c_ref.shape[0]
    p = g_ref[...]
    c = c_ref[...]
    cb = jnp.broadcast_to(c[:, None, :], (T, _K, _PW)).reshape(T * _K, _PW)
    d = jnp.abs(p - cb)
    h = p @ wps_ref[...] + cb @ wc_ref[...] + d @ wd_ref[...] + b0_ref[...]
    h = jnp.maximum(h, 0.0)
    h = jnp.maximum(h @ w1_ref[...] + b1_ref[...], 0.0)
    h = jnp.maximum(h @ w2_ref[...] + b2_ref[...], 0.0)
    x1 = _attn_pool(h, T, a1_ref, a2_ref, attb_ref)
    out_ref[...] = jnp.concatenate([x1, c], axis=-1)


def _layer_mid_body(g_ref, tp_ref, wf_ref, wps_ref, wc_ref, wd_ref, b_ref,
                    a1_ref, a2_ref, attb_ref, out_ref):
    T = tp_ref.shape[0]
    g = g_ref[...]
    f = g[:, 0:64]
    p = g[:, 64:80]
    tp = tp_ref[...]
    xprev = tp[:, 0:64]
    c = tp[:, 64:80]
    cb = jnp.broadcast_to(c[:, None, :], (T, _K, 16)).reshape(T * _K, 16)
    d = jnp.abs(p - cb)
    h = f @ wf_ref[...] + p @ wps_ref[...] + cb @ wc_ref[...] + d @ wd_ref[...]
    h = jnp.maximum(h + b_ref[...], 0.0)
    x = _attn_pool(h, T, a1_ref, a2_ref, attb_ref) + xprev
    out_ref[...] = jnp.concatenate([x, c], axis=-1)


def _layer_last_body(g_ref, t2_ref, t1_ref, wf_ref, wps_ref, wc_ref, wd_ref,
                     b_ref, a1_ref, a2_ref, attb_ref,
                     oa_ref, ob_ref, oc_ref, ob0_ref, ow1_ref, ob1_ref,
                     ow2_ref, ob2_ref, mu_ref, sg_ref):
    T = t2_ref.shape[0]
    g = g_ref[...]
    f = g[:, 0:64]
    p = g[:, 64:80]
    t2 = t2_ref[...]
    x2 = t2[:, 0:64]
    c = t2[:, 64:80]
    cb = jnp.broadcast_to(c[:, None, :], (T, _K, 16)).reshape(T * _K, 16)
    d = jnp.abs(p - cb)
    h = f @ wf_ref[...] + p @ wps_ref[...] + cb @ wc_ref[...] + d @ wd_ref[...]
    h = jnp.maximum(h + b_ref[...], 0.0)
    x3 = _attn_pool(h, T, a1_ref, a2_ref, attb_ref) + x2
    x1 = t1_ref[...][:, 0:64]
    h0 = jnp.maximum(x1 @ oa_ref[...] + x2 @ ob_ref[...] + x3 @ oc_ref[...]
                     + ob0_ref[...], 0.0)
    h1 = jnp.maximum(h0 @ ow1_ref[...] + ob1_ref[...], 0.0)
    ms = h1 @ ow2_ref[...] + ob2_ref[...]
    mu_ref[...] = ms[:, 0:16]
    sg_ref[...] = jnp.exp(ms[:, 16:32])


def _full_spec(shape):
    return pl.BlockSpec(shape, lambda i: tuple(0 for _ in shape))


def _row_spec(rows, cols):
    return pl.BlockSpec((rows, cols), lambda i: (i, 0))


def _pad16(w):
    return jnp.concatenate(
        [w, jnp.zeros((16 - w.shape[0], w.shape[1]), w.dtype)], axis=0)


def kernel(input, knn_idx_list,
           gc0_W0, gc0_b0, gc0_W1, gc0_b1, gc0_W2, gc0_b2, gc0_attW, gc0_attb,
           gc1_W0, gc1_b0, gc1_attW, gc1_attb,
           gc2_W0, gc2_b0, gc2_attW, gc2_attb,
           out_W0, out_b0, out_W1, out_b1, out_W2, out_b2):
    B, N, _ = input.shape
    grid = N // _T
    inp = input.reshape(N, 3)
    idx = knn_idx_list.astype(jnp.int32).reshape(3, N * _K)
    inp_pad = jnp.concatenate([inp, jnp.zeros((N, _PW - 3), jnp.float32)],
                              axis=1)

    def att_parts(attW, attb):
        return (attW[0:64, 0].reshape(1, 64), attW[64:128, 0].reshape(1, 64),
                attb.reshape(1, 1))

    w0 = gc0_W0
    l0_args = (
        _pad16(w0[0:3] + w0[3:6] + w0[6:9]), _pad16(-w0[6:9]),
        _pad16(w0[9:12]), gc0_b0.reshape(1, 16),
        gc0_W1, gc0_b1.reshape(1, 32), gc0_W2, gc0_b2.reshape(1, 64),
        *att_parts(gc0_attW, gc0_attb),
    )
    g0 = _sc_gather(inp_pad, idx[0])
    t1 = pl.pallas_call(
        _layer0_body,
        grid=(grid,),
        in_specs=[_row_spec(_T * _K, _PW), _row_spec(_T, _PW)]
        + [_full_spec(a.shape) for a in l0_args],
        out_specs=_row_spec(_T, _TW),
        out_shape=jax.ShapeDtypeStruct((N, _TW), jnp.float32),
    )(g0, inp_pad, *l0_args)

    def mid_args(W, b, attW, attb):
        return (W[0:64], _pad16(W[64:67] + W[67:70]), _pad16(-W[67:70]),
                _pad16(W[70:73]), b.reshape(1, 64), *att_parts(attW, attb))

    l1_args = mid_args(gc1_W0, gc1_b0, gc1_attW, gc1_attb)
    g1 = _sc_gather(t1, idx[1])
    t2 = pl.pallas_call(
        _layer_mid_body,
        grid=(grid,),
        in_specs=[_row_spec(_T * _K, _TW), _row_spec(_T, _TW)]
        + [_full_spec(a.shape) for a in l1_args],
        out_specs=_row_spec(_T, _TW),
        out_shape=jax.ShapeDtypeStruct((N, _TW), jnp.float32),
    )(g1, t1, *l1_args)

    l2_args = mid_args(gc2_W0, gc2_b0, gc2_attW, gc2_attb) + (
        out_W0[0:64], out_W0[64:128], out_W0[128:192], out_b0.reshape(1, 64),
        out_W1, out_b1.reshape(1, 64), out_W2, out_b2.reshape(1, 32),
    )
    g2 = _sc_gather(t2, idx[2])
    mu, sg = pl.pallas_call(
        _layer_last_body,
        grid=(grid,),
        in_specs=[_row_spec(_T * _K, _TW), _row_spec(_T, _TW),
                  _row_spec(_T, _TW)]
        + [_full_spec(a.shape) for a in l2_args],
        out_specs=[_row_spec(_T, 16), _row_spec(_T, 16)],
        out_shape=[jax.ShapeDtypeStruct((N, 16), jnp.float32),
                   jax.ShapeDtypeStruct((N, 16), jnp.float32)],
    )(g2, t2, t1, *l2_args)

    return mu.reshape(B, N, 16), sg.reshape(B, N, 16)


# speedup vs baseline: 7.2811x; 7.2811x over previous
import functools

import jax
import jax.numpy as jnp
from jax import lax
from jax.experimental import pallas as pl
from jax.experimental.pallas import tpu as pltpu
from jax.experimental.pallas import tpu_sc as plsc

_K = 16
_TW = 128
_T = 1000
_CH = 128
_NW = 32


@functools.lru_cache(maxsize=None)
def _sc_gather_fn(V, D, Bn):
    n_chunks = Bn // _CH
    per_w = (n_chunks + _NW - 1) // _NW
    mesh = plsc.VectorSubcoreMesh(core_axis_name="c", subcore_axis_name="s")

    @functools.partial(
        pl.kernel,
        mesh=mesh,
        out_type=jax.ShapeDtypeStruct((Bn, D), jnp.float32),
        scratch_types=[
            pltpu.VMEM((_CH,), jnp.int32),
            pltpu.VMEM((_CH, D), jnp.float32),
            pltpu.SemaphoreType.DMA,
        ],
    )
    def gk(table_hbm, idx_hbm, out_hbm, idx_v, rows_v, sem):
        wid = lax.axis_index("s") * 2 + lax.axis_index("c")

        def body(t, carry):
            cidx = wid + t * _NW

            @pl.when(cidx < n_chunks)
            def _():
                base = cidx * _CH
                pltpu.sync_copy(idx_hbm.at[pl.ds(base, _CH)], idx_v)
                pltpu.async_copy(table_hbm.at[idx_v], rows_v, sem).wait()
                pltpu.sync_copy(rows_v, out_hbm.at[pl.ds(base, _CH)])

            return carry

        lax.fori_loop(0, per_w, body, 0)

    return gk


def _sc_gather(table, idx_flat):
    V, D = table.shape
    return _sc_gather_fn(V, D, idx_flat.shape[0])(table, idx_flat)


def _leaky(x):
    return jnp.where(x > 0, x, 0.2 * x)


def _elu(x):
    return jnp.where(x > 0, x, jnp.exp(jnp.minimum(x, 0.0)) - 1.0)


def _attn_pool(h, T, a1_ref, a2_ref, attb_ref):
    h3 = h.reshape(T, _K, 64)
    a1 = a1_ref[...].reshape(1, 1, 64)
    a2 = a2_ref[...]
    l1 = jnp.sum(h3 * a1, axis=-1)
    hc = h3[:, 0, :]
    l2 = jnp.sum(hc * a2, axis=-1, keepdims=True)
    logit = _leaky(l1 + l2 + attb_ref[0, 0])
    m = jnp.max(logit, axis=-1, keepdims=True)
    e = jnp.exp(logit - m)
    a = e / jnp.sum(e, axis=-1, keepdims=True)
    return _elu(jnp.sum(h3 * a[:, :, None], axis=1))


def _layer0_body(g_ref, t0_ref, wps_ref, wc_ref, wd_ref, b0_ref,
                 w1_ref, b1_ref, w2_ref, b2_ref, a1_ref, a2_ref, attb_ref,
                 out_ref):
    T = t0_ref.shape[0]
    p = g_ref[:, 64:80]
    c = t0_ref[:, 64:80]
    cb = jnp.broadcast_to(c[:, None, :], (T, _K, 16)).reshape(T * _K, 16)
    d = jnp.abs(p - cb)
    h = p @ wps_ref[...] + cb @ wc_ref[...] + d @ wd_ref[...] + b0_ref[...]
    h = jnp.maximum(h, 0.0)
    h = jnp.maximum(h @ w1_ref[...] + b1_ref[...], 0.0)
    h = jnp.maximum(h @ w2_ref[...] + b2_ref[...], 0.0)
    x1 = _attn_pool(h, T, a1_ref, a2_ref, attb_ref)
    out_ref[...] = jnp.concatenate(
        [x1, c, jnp.zeros((T, _TW - 80), jnp.float32)], axis=-1)


def _layer_mid_body(g_ref, tp_ref, wf_ref, wps_ref, wc_ref, wd_ref, b_ref,
                    a1_ref, a2_ref, attb_ref, out_ref):
    T = tp_ref.shape[0]
    f = g_ref[:, 0:64]
    p = g_ref[:, 64:80]
    xprev = tp_ref[:, 0:64]
    c = tp_ref[:, 64:80]
    cb = jnp.broadcast_to(c[:, None, :], (T, _K, 16)).reshape(T * _K, 16)
    d = jnp.abs(p - cb)
    h = f @ wf_ref[...] + p @ wps_ref[...] + cb @ wc_ref[...] + d @ wd_ref[...]
    h = jnp.maximum(h + b_ref[...], 0.0)
    x = _attn_pool(h, T, a1_ref, a2_ref, attb_ref) + xprev
    out_ref[...] = jnp.concatenate(
        [x, c, jnp.zeros((T, _TW - 80), jnp.float32)], axis=-1)


def _layer_last_body(g_ref, t2_ref, t1_ref, wf_ref, wps_ref, wc_ref, wd_ref,
                     b_ref, a1_ref, a2_ref, attb_ref,
                     oa_ref, ob_ref, oc_ref, ob0_ref, ow1_ref, ob1_ref,
                     ow2_ref, ob2_ref, mu_ref, sg_ref):
    T = t2_ref.shape[0]
    f = g_ref[:, 0:64]
    p = g_ref[:, 64:80]
    x2 = t2_ref[:, 0:64]
    c = t2_ref[:, 64:80]
    cb = jnp.broadcast_to(c[:, None, :], (T, _K, 16)).reshape(T * _K, 16)
    d = jnp.abs(p - cb)
    h = f @ wf_ref[...] + p @ wps_ref[...] + cb @ wc_ref[...] + d @ wd_ref[...]
    h = jnp.maximum(h + b_ref[...], 0.0)
    x3 = _attn_pool(h, T, a1_ref, a2_ref, attb_ref) + x2
    x1 = t1_ref[:, 0:64]
    h0 = jnp.maximum(x1 @ oa_ref[...] + x2 @ ob_ref[...] + x3 @ oc_ref[...]
                     + ob0_ref[...], 0.0)
    h1 = jnp.maximum(h0 @ ow1_ref[...] + ob1_ref[...], 0.0)
    ms = h1 @ ow2_ref[...] + ob2_ref[...]
    mu_ref[...] = ms[:, 0:16]
    sg_ref[...] = jnp.exp(ms[:, 16:32])


def _full_spec(shape):
    return pl.BlockSpec(shape, lambda i: tuple(0 for _ in shape))


def _row_spec(rows, cols):
    return pl.BlockSpec((rows, cols), lambda i: (i, 0))


def _pad16(w):
    return jnp.concatenate(
        [w, jnp.zeros((16 - w.shape[0], w.shape[1]), w.dtype)], axis=0)


def kernel(input, knn_idx_list,
           gc0_W0, gc0_b0, gc0_W1, gc0_b1, gc0_W2, gc0_b2, gc0_attW, gc0_attb,
           gc1_W0, gc1_b0, gc1_attW, gc1_attb,
           gc2_W0, gc2_b0, gc2_attW, gc2_attb,
           out_W0, out_b0, out_W1, out_b1, out_W2, out_b2):
    B, N, _ = input.shape
    grid = N // _T
    inp = input.reshape(N, 3)
    idx = knn_idx_list.astype(jnp.int32).reshape(3, N * _K)
    t0 = jnp.concatenate(
        [jnp.zeros((N, 64), jnp.float32), inp,
         jnp.zeros((N, _TW - 67), jnp.float32)], axis=1)

    def att_parts(attW, attb):
        return (attW[0:64, 0].reshape(1, 64), attW[64:128, 0].reshape(1, 64),
                attb.reshape(1, 1))

    w0 = gc0_W0
    l0_args = (
        _pad16(w0[0:3] + w0[3:6] + w0[6:9]), _pad16(-w0[6:9]),
        _pad16(w0[9:12]), gc0_b0.reshape(1, 16),
        gc0_W1, gc0_b1.reshape(1, 32), gc0_W2, gc0_b2.reshape(1, 64),
        *att_parts(gc0_attW, gc0_attb),
    )
    g0 = _sc_gather(t0, idx[0])
    t1 = pl.pallas_call(
        _layer0_body,
        grid=(grid,),
        in_specs=[_row_spec(_T * _K, _TW), _row_spec(_T, _TW)]
        + [_full_spec(a.shape) for a in l0_args],
        out_specs=_row_spec(_T, _TW),
        out_shape=jax.ShapeDtypeStruct((N, _TW), jnp.float32),
    )(g0, t0, *l0_args)

    def mid_args(W, b, attW, attb):
        return (W[0:64], _pad16(W[64:67] + W[67:70]), _pad16(-W[67:70]),
                _pad16(W[70:73]), b.reshape(1, 64), *att_parts(attW, attb))

    l1_args = mid_args(gc1_W0, gc1_b0, gc1_attW, gc1_attb)
    g1 = _sc_gather(t1, idx[1])
    t2 = pl.pallas_call(
        _layer_mid_body,
        grid=(grid,),
        in_specs=[_row_spec(_T * _K, _TW), _row_spec(_T, _TW)]
        + [_full_spec(a.shape) for a in l1_args],
        out_specs=_row_spec(_T, _TW),
        out_shape=jax.ShapeDtypeStruct((N, _TW), jnp.float32),
    )(g1, t1, *l1_args)

    l2_args = mid_args(gc2_W0, gc2_b0, gc2_attW, gc2_attb) + (
        out_W0[0:64], out_W0[64:128], out_W0[128:192], out_b0.reshape(1, 64),
        out_W1, out_b1.reshape(1, 64), out_W2, out_b2.reshape(1, 32),
    )
    g2 = _sc_gather(t2, idx[2])
    mu, sg = pl.pallas_call(
        _layer_last_body,
        grid=(grid,),
        in_specs=[_row_spec(_T * _K, _TW), _row_spec(_T, _TW),
                  _row_spec(_T, _TW)]
        + [_full_spec(a.shape) for a in l2_args],
        out_specs=[_row_spec(_T, 16), _row_spec(_T, 16)],
        out_shape=[jax.ShapeDtypeStruct((N, 16), jnp.float32),
                   jax.ShapeDtypeStruct((N, 16), jnp.float32)],
    )(g2, t2, t1, *l2_args)

    return mu.reshape(B, N, 16), sg.reshape(B, N, 16)
